# baseline (speedup 1.0000x reference)
import functools

import jax
import jax.numpy as jnp
from jax import lax
from jax.experimental import pallas as pl
from jax.experimental.pallas import tpu as pltpu
from jax.experimental.pallas import tpu_sc as plsc

_B, _N, _C = 64, 2304, 1024
_HID = 256
_K = 256
_NCHUNK = 256
_NBLK = _N // _NCHUNK


def _score_topk_kernel(x_ref, w1_ref, b1_ref, w2_ref, idx_ref, s_col, s_row):
    b = pl.program_id(0)
    n = pl.program_id(1)

    x = x_ref[0].astype(jnp.bfloat16)
    acc = jnp.dot(
        x, w1_ref[...].astype(jnp.bfloat16), preferred_element_type=jnp.float32
    )
    h = jnp.maximum(acc + b1_ref[...], 0.0)
    s_t = lax.dot_general(
        w2_ref[...].astype(jnp.bfloat16),
        jnp.swapaxes(h, 0, 1).astype(jnp.bfloat16),
        (((0,), (0,)), ((), ())),
        preferred_element_type=jnp.float32,
    )

    s_col[pl.ds(n * _NCHUNK, _NCHUNK), :] = jnp.swapaxes(s_t, 0, 1)
    s_row[:, pl.ds(n * _NCHUNK, _NCHUNK)] = s_t

    @pl.when(n == _NBLK - 1)
    def _rank_and_select():
        srow = s_row[...]
        j_iota = lax.broadcasted_iota(jnp.int32, (1, _N), 1)
        r_iota = lax.broadcasted_iota(jnp.int32, (1, _K), 1)
        acc = jnp.zeros((1, _K), dtype=jnp.int32)
        for c in range(_NBLK):
            s_i = s_col[pl.ds(c * _NCHUNK, _NCHUNK), :]
            i_iota = (
                lax.broadcasted_iota(jnp.int32, (_NCHUNK, 1), 0) + c * _NCHUNK
            )
            beats = (srow > s_i) | ((srow == s_i) & (j_iota < i_iota))
            rank = jnp.sum(beats.astype(jnp.int32), axis=1, keepdims=True)
            onehot = rank == r_iota
            sel = jnp.where(onehot, jnp.broadcast_to(i_iota, (_NCHUNK, _K)), 0)
            acc += jnp.sum(sel, axis=0, keepdims=True)
        idx_ref[...] = (acc + b * _N).reshape(1, 1, _K)


def _score_topk(x, w1, b1, w2):
    return pl.pallas_call(
        _score_topk_kernel,
        grid=(_B, _NBLK),
        in_specs=[
            pl.BlockSpec((1, _NCHUNK, _C), lambda b, n: (b, n, 0)),
            pl.BlockSpec((_C, _HID), lambda b, n: (0, 0)),
            pl.BlockSpec((1, _HID), lambda b, n: (0, 0)),
            pl.BlockSpec((_HID, 1), lambda b, n: (0, 0)),
        ],
        out_specs=pl.BlockSpec((1, 1, _K), lambda b, n: (b, 0, 0)),
        out_shape=jax.ShapeDtypeStruct((_B, 1, _K), jnp.int32),
        scratch_shapes=[
            pltpu.VMEM((_N, 1), jnp.float32),
            pltpu.VMEM((1, _N), jnp.float32),
        ],
    )(x, w1, b1, w2)


_R = _B * _K
_NW = 32
_ROWS_PER_W = _R // _NW
_GCHUNK = 64
_NGCH = _ROWS_PER_W // _GCHUNK


def _make_gather():
    mesh = plsc.VectorSubcoreMesh(core_axis_name="c", subcore_axis_name="s")

    @functools.partial(
        pl.kernel,
        mesh=mesh,
        out_type=jax.ShapeDtypeStruct((_R, _C), jnp.float32),
        scratch_types=[
            pltpu.VMEM((_GCHUNK,), jnp.int32),
            pltpu.VMEM((_GCHUNK, _C), jnp.float32),
            pltpu.SemaphoreType.DMA,
        ],
    )
    def _gather(x_hbm, idx_hbm, out_hbm, idx_v, rows_v, sem):
        wid = lax.axis_index("s") * 2 + lax.axis_index("c")
        base = wid * _ROWS_PER_W
        for ch in range(_NGCH):
            off = base + ch * _GCHUNK
            pltpu.sync_copy(idx_hbm.at[pl.ds(off, _GCHUNK)], idx_v)
            pltpu.async_copy(x_hbm.at[idx_v], rows_v, sem).wait()
            pltpu.sync_copy(rows_v, out_hbm.at[pl.ds(off, _GCHUNK)])

    return _gather


def kernel(vit_features, W1, b1, W2, b2):
    del b2
    idx = _score_topk(vit_features, W1, b1.reshape(1, _HID), W2)
    flat_idx = idx.reshape(_R)
    table = vit_features.reshape(_B * _N, _C)
    rows = _make_gather()(table, flat_idx)
    return rows.reshape(_B, _K, _C)
